# baseline (speedup 1.0000x reference)
import functools

import jax
import jax.numpy as jnp
from jax import lax
from jax.experimental import pallas as pl
from jax.experimental.pallas import tpu as pltpu
from jax.experimental.pallas import tpu_sc as plsc

_B, _T, _D, _H, _E = 2, 2048, 768, 768, 8
_NT = _B * _T
_NA = 2 * _NT
_TILE = 256
_NP = _NA + _E * _TILE
_NTILES = _NP // _TILE
_RCH = 512

_NW = 32
_APW = _NA // _NW
_DCH = 128
_CCH = 32


def _routing_body(x_ref, gw_ref, m_ref, pos_ref, w_ref, te_ref):
    logits = lax.dot_general(
        x_ref[...], gw_ref[...], (((1,), (1,)), ((), ())),
        preferred_element_type=jnp.float32)
    mx = jnp.max(logits, axis=-1, keepdims=True)
    p = jnp.exp(logits - mx)
    p = p / jnp.sum(p, axis=-1, keepdims=True)
    p = jnp.where(m_ref[...] > 0, 0.0, p)

    iota_e = lax.broadcasted_iota(jnp.int32, (_NT, _E), 1)
    i1 = jnp.argmax(p, axis=-1)
    oh1 = (iota_e == i1[:, None]).astype(jnp.float32)
    i2 = jnp.argmax(jnp.where(oh1 > 0, -1.0, p), axis=-1)
    oh2 = (iota_e == i2[:, None]).astype(jnp.float32)

    g1 = jnp.sum(p * oh1, axis=-1, keepdims=True)
    g2 = jnp.sum(p * oh2, axis=-1, keepdims=True)
    s = g1 + g2
    s = jnp.where(s == 0.0, 1.0, s)
    w_ref[0] = jnp.broadcast_to(g1 / s, (_NT, 16))
    w_ref[1] = jnp.broadcast_to(g2 / s, (_NT, 16))

    ir = lax.broadcasted_iota(jnp.int32, (_RCH, _RCH), 0)
    ic = lax.broadcasted_iota(jnp.int32, (_RCH, _RCH), 1)
    tri = (ir >= ic).astype(jnp.float32)

    carry = jnp.zeros((1, _E), jnp.float32)
    ranks = []
    for c in range(_NA // _RCH):
        oh = oh1 if c < _NT // _RCH else oh2
        r0 = (c % (_NT // _RCH)) * _RCH
        ohc = oh[r0:r0 + _RCH, :]
        incl = lax.dot_general(
            tri, ohc, (((1,), (0,)), ((), ())),
            preferred_element_type=jnp.float32) + carry
        ranks.append(jnp.sum(ohc * incl, axis=-1) - 1.0)
        carry = incl[_RCH - 1:_RCH, :]

    counts = carry
    ftile = jnp.float32(_TILE)
    aligned = jnp.floor((counts + (ftile - 1.0)) / ftile) * ftile
    ie1 = lax.broadcasted_iota(jnp.int32, (_E, _E), 0)
    ie2 = lax.broadcasted_iota(jnp.int32, (_E, _E), 1)
    tri_s = (ie1 < ie2).astype(jnp.float32)
    poff = lax.dot_general(
        aligned, tri_s, (((1,), (0,)), ((), ())),
        preferred_element_type=jnp.float32)
    pend = poff + aligned

    for c in range(_NA // _RCH):
        oh = oh1 if c < _NT // _RCH else oh2
        r0 = (c % (_NT // _RCH)) * _RCH
        ohc = oh[r0:r0 + _RCH, :]
        base = jnp.sum(ohc * poff, axis=-1)
        slot = c // (_NT // _RCH)
        posc = (base + ranks[c]).astype(jnp.int32)
        pos_ref[slot, pl.ds(r0, _RCH)] = posc

    it = lax.broadcasted_iota(jnp.int32, (1, _NTILES), 1).astype(
        jnp.float32) * ftile
    te = jnp.zeros((1, _NTILES), jnp.float32)
    for e in range(_E):
        te = te + (it >= pend[0, e]).astype(jnp.float32)
    te_ref[...] = jnp.minimum(te, float(_E - 1)).astype(jnp.int32)


def _routing(xf, gate_w, maskf):
    return pl.pallas_call(
        _routing_body,
        in_specs=[
            pl.BlockSpec((_NT, _D), lambda: (0, 0)),
            pl.BlockSpec((_E, _D), lambda: (0, 0)),
            pl.BlockSpec((_NT, 1), lambda: (0, 0)),
        ],
        out_specs=[
            pl.BlockSpec((2, _NT), lambda: (0, 0)),
            pl.BlockSpec((2, _NT, 16), lambda: (0, 0, 0)),
            pl.BlockSpec((1, _NTILES), lambda: (0, 0)),
        ],
        out_shape=[
            jax.ShapeDtypeStruct((2, _NT), jnp.int32),
            jax.ShapeDtypeStruct((2, _NT, 16), jnp.float32),
            jax.ShapeDtypeStruct((1, _NTILES), jnp.int32),
        ],
    )(xf, gate_w, maskf)


def _dispatch_body(x_hbm, pos_hbm, xs_hbm, idx_v, rows_v, sem):
    wid = lax.axis_index("s") * 2 + lax.axis_index("c")
    for c in range(_APW // _DCH):
        base = wid * _APW + c * _DCH
        base = pl.multiple_of(base, _DCH)
        tok = lax.rem(base, _NT)
        tok = pl.multiple_of(tok, _DCH)
        pltpu.sync_copy(pos_hbm.at[pl.ds(base, _DCH)], idx_v)
        pltpu.sync_copy(x_hbm.at[pl.ds(tok, _DCH)], rows_v)
        pltpu.async_copy(rows_v, xs_hbm.at[idx_v], sem).wait()


def _dispatch(xf, posf):
    mesh = plsc.VectorSubcoreMesh(core_axis_name="c", subcore_axis_name="s")
    f = pl.kernel(
        _dispatch_body,
        out_type=jax.ShapeDtypeStruct((_NP, _D), jnp.float32),
        mesh=mesh,
        scratch_types=[
            pltpu.VMEM((_DCH,), jnp.int32),
            pltpu.VMEM((_DCH, _D), jnp.float32),
            pltpu.SemaphoreType.DMA,
        ],
    )
    return f(xf, posf)


def _ffn_body(te_ref, xs_ref, w1_ref, b1_ref, w2_ref, b2_ref, ys_ref):
    h = lax.dot_general(
        xs_ref[...], w1_ref[0], (((1,), (1,)), ((), ())),
        preferred_element_type=jnp.float32)
    h = jnp.maximum(h + b1_ref[0], 0.0)
    y = lax.dot_general(
        h, w2_ref[0], (((1,), (1,)), ((), ())),
        preferred_element_type=jnp.float32)
    ys_ref[...] = y + b2_ref[0]


def _ffn(te, xs, fc1_w, fc1_b, fc2_w, fc2_b):
    grid_spec = pltpu.PrefetchScalarGridSpec(
        num_scalar_prefetch=1,
        grid=(_NTILES,),
        in_specs=[
            pl.BlockSpec((_TILE, _D), lambda i, te: (i, 0)),
            pl.BlockSpec((1, _H, _D), lambda i, te: (te[i], 0, 0)),
            pl.BlockSpec((1, 1, _H), lambda i, te: (te[i], 0, 0)),
            pl.BlockSpec((1, _D, _H), lambda i, te: (te[i], 0, 0)),
            pl.BlockSpec((1, 1, _D), lambda i, te: (te[i], 0, 0)),
        ],
        out_specs=pl.BlockSpec((_TILE, _D), lambda i, te: (i, 0)),
    )
    return pl.pallas_call(
        _ffn_body,
        grid_spec=grid_spec,
        out_shape=jax.ShapeDtypeStruct((_NP, _D), jnp.float32),
    )(te, xs, fc1_w, fc1_b.reshape(_E, 1, _H), fc2_w,
      fc2_b.reshape(_E, 1, _D))


def _combine_body(ys_hbm, pos_hbm, w_hbm, out_hbm,
                  idx1_v, idx2_v, w1_v, w2_v, r1_v, r2_v, o_v, sem1, sem2):
    wid = lax.axis_index("s") * 2 + lax.axis_index("c")
    for c in range(_NT // _NW // _CCH):
        tok = wid * (_NT // _NW) + c * _CCH
        tok = pl.multiple_of(tok, _CCH)
        pltpu.sync_copy(pos_hbm.at[pl.ds(tok, _CCH)], idx1_v)
        pltpu.sync_copy(pos_hbm.at[pl.ds(_NT + tok, _CCH)], idx2_v)
        pltpu.sync_copy(w_hbm.at[pl.ds(tok, _CCH)], w1_v)
        pltpu.sync_copy(w_hbm.at[pl.ds(_NT + tok, _CCH)], w2_v)
        cp1 = pltpu.async_copy(ys_hbm.at[idx1_v], r1_v, sem1)
        cp2 = pltpu.async_copy(ys_hbm.at[idx2_v], r2_v, sem2)
        cp1.wait()
        cp2.wait()

        def body(i, _):
            w1s = w1_v[i, :]
            w2s = w2_v[i, :]
            for j in range(_D // 16):
                sl = pl.ds(j * 16, 16)
                o_v[i, sl] = w1s * r1_v[i, sl] + w2s * r2_v[i, sl]
            return 0

        lax.fori_loop(0, _CCH, body, 0)
        pltpu.sync_copy(o_v, out_hbm.at[pl.ds(tok, _CCH)])


def _combine(ys, posf, wf):
    mesh = plsc.VectorSubcoreMesh(core_axis_name="c", subcore_axis_name="s")
    f = pl.kernel(
        _combine_body,
        out_type=jax.ShapeDtypeStruct((_NT, _D), jnp.float32),
        mesh=mesh,
        scratch_types=[
            pltpu.VMEM((_CCH,), jnp.int32),
            pltpu.VMEM((_CCH,), jnp.int32),
            pltpu.VMEM((_CCH, 16), jnp.float32),
            pltpu.VMEM((_CCH, 16), jnp.float32),
            pltpu.VMEM((_CCH, _D), jnp.float32),
            pltpu.VMEM((_CCH, _D), jnp.float32),
            pltpu.VMEM((_CCH, _D), jnp.float32),
            pltpu.SemaphoreType.DMA,
            pltpu.SemaphoreType.DMA,
        ],
    )
    return f(ys, posf, wf)


def kernel(x, padding_mask, gate_w, fc1_w, fc1_b, fc2_w, fc2_b):
    xf = x.reshape(_NT, _D)
    maskf = padding_mask.reshape(_NT, 1).astype(jnp.float32)

    pos2, w2, te = _routing(xf, gate_w, maskf)
    posf = pos2.reshape(_NA)
    wf = w2.reshape(_NA, 16)

    xs = _dispatch(xf, posf)
    ys = _ffn(te.reshape(_NTILES), xs, fc1_w, fc1_b, fc2_w, fc2_b)
    out = _combine(ys, posf, wf)
    return out.reshape(_B, _T, _D)
